# Initial kernel scaffold; baseline (speedup 1.0000x reference)
import jax
import jax.numpy as jnp
from jax.experimental import pallas as pl


def kernel(x, occ_grid, occ_grid_binary, aabb):
    raise NotImplementedError("write your pallas kernel here")


# speedup vs baseline: 1.6205x; 1.6205x over previous
import functools

import jax
import jax.numpy as jnp
from jax import lax
from jax.experimental import pallas as pl
from jax.experimental.pallas import tpu as pltpu
from jax.experimental.pallas import tpu_sc as plsc

RES = (128, 128, 128)
NUM_CELLS = RES[0] * RES[1] * RES[2]
N_SAMPLES = 2097152

NC = 2
NS = 16
NW = NC * NS
PER_W = N_SAMPLES // NW
CHUNK = 8192
NCHUNK = PER_W // CHUNK
GROUPS = CHUNK // 16

_SIGN = -2147483648
_MANT = 2147483647


def _pack_body(occ_ref, bin_ref, out_ref):
    bits = lax.bitcast_convert_type(occ_ref[...], jnp.int32)
    out_ref[...] = bits | jnp.where(
        bin_ref[...], jnp.int32(_SIGN), jnp.int32(0))


def _pack_table(occ_grid, occ_grid_binary):
    occ2 = occ_grid.reshape(256, 8192)
    bin2 = occ_grid_binary.reshape(256, 8192)
    packed = pl.pallas_call(
        _pack_body,
        grid=(8,),
        in_specs=[
            pl.BlockSpec((32, 8192), lambda i: (i, 0)),
            pl.BlockSpec((32, 8192), lambda i: (i, 0)),
        ],
        out_specs=pl.BlockSpec((32, 8192), lambda i: (i, 0)),
        out_shape=jax.ShapeDtypeStruct((256, 8192), jnp.int32),
    )(occ2, bin2)
    return packed.reshape(-1)


def _sc_body(xs_hbm, ys_hbm, zs_hbm, packed_hbm, consts_hbm, occ_out,
             bin_out, xsbuf, ysbuf, zsbuf, idxbuf, selbuf, gbuf, obuf, bbuf,
             cbuf, sem):
    wid = lax.axis_index("s") * NC + lax.axis_index("c")

    pltpu.sync_copy(consts_hbm, cbuf)
    minx = cbuf[0]
    miny = cbuf[1]
    minz = cbuf[2]
    sclx = cbuf[3]
    scly = cbuf[4]
    sclz = cbuf[5]

    for c in range(NCHUNK):
        base = wid * PER_W + c * CHUNK
        pltpu.sync_copy(xs_hbm.at[pl.ds(base, CHUNK)], xsbuf)
        pltpu.sync_copy(ys_hbm.at[pl.ds(base, CHUNK)], ysbuf)
        pltpu.sync_copy(zs_hbm.at[pl.ds(base, CHUNK)], zsbuf)

        def _idx_grp(g, _):
            s = pl.ds(g * 16, 16)
            ax = xsbuf[s]
            ay = ysbuf[s]
            az = zsbuf[s]
            tx = (ax - minx) * sclx
            ty = (ay - miny) * scly
            tz = (az - minz) * sclz
            sel = ((tx > 0.0) & (tx < 128.0)
                   & (ty > 0.0) & (ty < 128.0)
                   & (tz > 0.0) & (tz < 128.0))
            cx = jnp.minimum(jnp.maximum(tx, 0.0), 127.0).astype(jnp.int32)
            cy = jnp.minimum(jnp.maximum(ty, 0.0), 127.0).astype(jnp.int32)
            cz = jnp.minimum(jnp.maximum(tz, 0.0), 127.0).astype(jnp.int32)
            idx = cx * 16384 + cy * 128 + cz
            idxbuf[s] = idx
            selbuf[s] = jnp.where(sel, jnp.int32(1), jnp.int32(0))
            return 0

        lax.fori_loop(0, GROUPS, _idx_grp, 0)

        pltpu.async_copy(packed_hbm.at[idxbuf], gbuf, sem).wait()

        def _post_grp(g, _):
            s = pl.ds(g * 16, 16)
            pv = gbuf[s]
            sv = selbuf[s]
            live = sv != 0
            occb = pv & jnp.int32(_MANT)
            binv = lax.shift_right_logical(pv, 31)
            obuf[s] = jnp.where(live, occb, jnp.int32(0))
            bbuf[s] = jnp.where(live, binv, jnp.int32(0))
            return 0

        lax.fori_loop(0, GROUPS, _post_grp, 0)

        pltpu.sync_copy(obuf, occ_out.at[pl.ds(base, CHUNK)])
        pltpu.sync_copy(bbuf, bin_out.at[pl.ds(base, CHUNK)])


_sc_gather = functools.partial(
    pl.kernel,
    mesh=plsc.VectorSubcoreMesh(core_axis_name="c", subcore_axis_name="s"),
    out_type=[
        jax.ShapeDtypeStruct((N_SAMPLES,), jnp.int32),
        jax.ShapeDtypeStruct((N_SAMPLES,), jnp.int32),
    ],
    scratch_types=[
        pltpu.VMEM((CHUNK,), jnp.float32),
        pltpu.VMEM((CHUNK,), jnp.float32),
        pltpu.VMEM((CHUNK,), jnp.float32),
        pltpu.VMEM((CHUNK,), jnp.int32),
        pltpu.VMEM((CHUNK,), jnp.int32),
        pltpu.VMEM((CHUNK,), jnp.int32),
        pltpu.VMEM((CHUNK,), jnp.int32),
        pltpu.VMEM((CHUNK,), jnp.int32),
        pltpu.VMEM((6, 16), jnp.float32),
        pltpu.SemaphoreType.DMA,
    ],
)(_sc_body)


def kernel(x, occ_grid, occ_grid_binary, aabb):
    packed = _pack_table(occ_grid, occ_grid_binary)

    bb_min = aabb[:3]
    scl = jnp.float32(RES[0]) / (aabb[3:] - aabb[:3])
    consts = jnp.concatenate(
        [jnp.broadcast_to(bb_min[:, None], (3, 16)),
         jnp.broadcast_to(scl[:, None], (3, 16))], axis=0)

    xt = x.T
    occ_bits, occs_bin = _sc_gather(xt[0], xt[1], xt[2], packed, consts)
    occs = lax.bitcast_convert_type(occ_bits, jnp.float32)
    return occs, occs_bin.astype(jnp.bool_)


# speedup vs baseline: 2.0076x; 1.2389x over previous
import functools

import jax
import jax.numpy as jnp
from jax import lax
from jax.experimental import pallas as pl
from jax.experimental.pallas import tpu as pltpu
from jax.experimental.pallas import tpu_sc as plsc

RES = (128, 128, 128)
NUM_CELLS = RES[0] * RES[1] * RES[2]
N_SAMPLES = 2097152

NC = 2
NS = 16
NW = NC * NS
PER_W = N_SAMPLES // NW
CHUNK = 8192
NCHUNK = PER_W // CHUNK
GROUPS = CHUNK // 16

PACK_ROWS = 264
PACK_COLS = 8192

_SIGN = -2147483648
_MANT = 2147483647


def _pack_body(occ_ref, bin_ref, out_ref):
    i = pl.program_id(0)

    @pl.when(i < 32)
    def _pack():
        bits = lax.bitcast_convert_type(occ_ref[...], jnp.int32)
        out_ref[...] = bits | jnp.where(
            bin_ref[...], jnp.int32(_SIGN), jnp.int32(0))

    @pl.when(i == 32)
    def _zero():
        out_ref[...] = jnp.zeros_like(out_ref)


def _pack_table(occ_grid, occ_grid_binary):
    occ2 = occ_grid.reshape(256, PACK_COLS)
    bin2 = occ_grid_binary.reshape(256, PACK_COLS)
    packed = pl.pallas_call(
        _pack_body,
        grid=(33,),
        in_specs=[
            pl.BlockSpec((8, PACK_COLS), lambda i: (jnp.minimum(i, 31), 0)),
            pl.BlockSpec((8, PACK_COLS), lambda i: (jnp.minimum(i, 31), 0)),
        ],
        out_specs=pl.BlockSpec((8, PACK_COLS), lambda i: (i, 0)),
        out_shape=jax.ShapeDtypeStruct((PACK_ROWS, PACK_COLS), jnp.int32),
    )(occ2, bin2)
    return packed.reshape(-1)


def _sc_body(xs_hbm, ys_hbm, zs_hbm, packed_hbm, consts_hbm, occ_out,
             bin_out,
             xs0, ys0, zs0, xs1, ys1, zs1, idx0, idx1, g0, g1, o0, o1,
             b0, b1, cbuf,
             sin0, sin1, sg0, sg1, so0, so1):
    wid = lax.axis_index("s") * NC + lax.axis_index("c")

    pltpu.sync_copy(consts_hbm, cbuf)
    minx = cbuf[0]
    miny = cbuf[1]
    minz = cbuf[2]
    sclx = cbuf[3]
    scly = cbuf[4]
    sclz = cbuf[5]

    xsb = (xs0, xs1)
    ysb = (ys0, ys1)
    zsb = (zs0, zs1)
    idxb = (idx0, idx1)
    gb = (g0, g1)
    ob = (o0, o1)
    bb = (b0, b1)
    sin = (sin0, sin1)
    sg = (sg0, sg1)
    so = (so0, so1)

    def issue_in(c):
        par = c % 2
        base = wid * PER_W + c * CHUNK
        return (
            pltpu.async_copy(xs_hbm.at[pl.ds(base, CHUNK)], xsb[par],
                             sin[par]),
            pltpu.async_copy(ys_hbm.at[pl.ds(base, CHUNK)], ysb[par],
                             sin[par]),
            pltpu.async_copy(zs_hbm.at[pl.ds(base, CHUNK)], zsb[par],
                             sin[par]),
        )

    def compute_idx(c):
        par = c % 2
        xsv, ysv, zsv, idxv = xsb[par], ysb[par], zsb[par], idxb[par]

        def _grp(g, _):
            s = pl.ds(g * 16, 16)
            tx = (xsv[s] - minx) * sclx
            ty = (ysv[s] - miny) * scly
            tz = (zsv[s] - minz) * sclz
            lo = jnp.minimum(jnp.minimum(tx, ty), tz)
            hi = jnp.maximum(jnp.maximum(tx, ty), tz)
            sel = (lo > 0.0) & (hi < 128.0)
            cx = tx.astype(jnp.int32)
            cy = ty.astype(jnp.int32)
            cz = tz.astype(jnp.int32)
            idx = cx * 16384 + cy * 128 + cz
            idxv[s] = jnp.where(sel, idx, jnp.int32(NUM_CELLS))
            return 0

        lax.fori_loop(0, GROUPS, _grp, 0)

    def unpack(c):
        par = c % 2
        gv, ov, bv = gb[par], ob[par], bb[par]

        def _grp(g, _):
            s = pl.ds(g * 16, 16)
            pv = gv[s]
            ov[s] = pv & jnp.int32(_MANT)
            bv[s] = lax.shift_right_logical(pv, 31)
            return 0

        lax.fori_loop(0, GROUPS, _grp, 0)

    def issue_out(c):
        par = c % 2
        base = wid * PER_W + c * CHUNK
        return (
            pltpu.async_copy(ob[par], occ_out.at[pl.ds(base, CHUNK)],
                             so[par]),
            pltpu.async_copy(bb[par], bin_out.at[pl.ds(base, CHUNK)],
                             so[par]),
        )

    pending_in = {0: issue_in(0)}
    pending_g = {}
    pending_out = {}

    for c in range(NCHUNK):
        if c + 1 < NCHUNK:
            pending_in[c + 1] = issue_in(c + 1)
        for h in pending_in.pop(c):
            h.wait()
        compute_idx(c)
        pending_g[c] = pltpu.async_copy(
            packed_hbm.at[idxb[c % 2]], gb[c % 2], sg[c % 2])
        if c - 1 >= 0:
            pending_g.pop(c - 1).wait()
            if c - 3 >= 0:
                for h in pending_out.pop(c - 3):
                    h.wait()
            unpack(c - 1)
            pending_out[c - 1] = issue_out(c - 1)

    last = NCHUNK - 1
    pending_g.pop(last).wait()
    if last - 2 >= 0:
        for h in pending_out.pop(last - 2):
            h.wait()
    unpack(last)
    pending_out[last] = issue_out(last)
    for c in sorted(pending_out):
        for h in pending_out[c]:
            h.wait()


_sc_gather = functools.partial(
    pl.kernel,
    mesh=plsc.VectorSubcoreMesh(core_axis_name="c", subcore_axis_name="s"),
    out_type=[
        jax.ShapeDtypeStruct((N_SAMPLES,), jnp.int32),
        jax.ShapeDtypeStruct((N_SAMPLES,), jnp.int32),
    ],
    scratch_types=[
        pltpu.VMEM((CHUNK,), jnp.float32),
        pltpu.VMEM((CHUNK,), jnp.float32),
        pltpu.VMEM((CHUNK,), jnp.float32),
        pltpu.VMEM((CHUNK,), jnp.float32),
        pltpu.VMEM((CHUNK,), jnp.float32),
        pltpu.VMEM((CHUNK,), jnp.float32),
        pltpu.VMEM((CHUNK,), jnp.int32),
        pltpu.VMEM((CHUNK,), jnp.int32),
        pltpu.VMEM((CHUNK,), jnp.int32),
        pltpu.VMEM((CHUNK,), jnp.int32),
        pltpu.VMEM((CHUNK,), jnp.int32),
        pltpu.VMEM((CHUNK,), jnp.int32),
        pltpu.VMEM((CHUNK,), jnp.int32),
        pltpu.VMEM((CHUNK,), jnp.int32),
        pltpu.VMEM((6, 16), jnp.float32),
        pltpu.SemaphoreType.DMA,
        pltpu.SemaphoreType.DMA,
        pltpu.SemaphoreType.DMA,
        pltpu.SemaphoreType.DMA,
        pltpu.SemaphoreType.DMA,
        pltpu.SemaphoreType.DMA,
    ],
)(_sc_body)


def kernel(x, occ_grid, occ_grid_binary, aabb):
    packed = _pack_table(occ_grid, occ_grid_binary)

    bb_min = aabb[:3]
    scl = jnp.float32(RES[0]) / (aabb[3:] - aabb[:3])
    consts = jnp.concatenate(
        [jnp.broadcast_to(bb_min[:, None], (3, 16)),
         jnp.broadcast_to(scl[:, None], (3, 16))], axis=0)

    xt = x.T
    occ_bits, occs_bin = _sc_gather(xt[0], xt[1], xt[2], packed, consts)
    occs = lax.bitcast_convert_type(occ_bits, jnp.float32)
    return occs, occs_bin.astype(jnp.bool_)
